# Initial kernel scaffold; baseline (speedup 1.0000x reference)
import jax
import jax.numpy as jnp
from jax.experimental import pallas as pl


def kernel(heat_pred, paf_pred):
    raise NotImplementedError("write your pallas kernel here")


# speedup vs baseline: 1.7637x; 1.7637x over previous
import functools

import jax
import jax.numpy as jnp
import numpy as np
from jax import lax
from jax.experimental import pallas as pl
from jax.experimental.pallas import tpu as pltpu
from jax.experimental.pallas import tpu_sc as plsc

_SKEL = np.array(
    [[15, 13], [13, 11], [16, 14], [14, 12], [11, 12], [5, 11], [6, 12],
     [5, 6], [5, 7], [6, 8], [7, 9], [8, 10], [1, 2], [0, 1], [0, 2],
     [1, 3], [2, 4], [3, 5], [4, 6]], dtype=np.int32)

_PEAK_THRESH = 0.1
_PAF_SCORE_THRESH = 0.05
_MAX_PEAKS = 20
_N_SAMPLES = 10
_NEG = -1e9
_H = 128
_W = 128
_NINF = float("-inf")


def _peaks_body(h_ref, px_ref, py_ref, scr_ref, val_ref):
    h = h_ref[0]
    col_ninf = jnp.full((_H, 1), _NINF, jnp.float32)
    s_l = jnp.concatenate([h[:, 1:], col_ninf], axis=1)
    s_r = jnp.concatenate([col_ninf, h[:, :-1]], axis=1)
    rmax = jnp.maximum(h, jnp.maximum(s_l, s_r))
    row_ninf = jnp.full((1, _W), _NINF, jnp.float32)
    s_u = jnp.concatenate([rmax[1:, :], row_ninf], axis=0)
    s_d = jnp.concatenate([row_ninf, rmax[:-1, :]], axis=0)
    pooled = jnp.maximum(rmax, jnp.maximum(s_u, s_d))

    is_peak = (h == pooled) & (h > _PEAK_THRESH)
    masked = jnp.where(is_peak, h, jnp.float32(_NEG))

    ri = lax.broadcasted_iota(jnp.int32, (_H, _W), 0)
    ci = lax.broadcasted_iota(jnp.int32, (_H, _W), 1)
    lin = ri * _W + ci

    li20 = lax.broadcasted_iota(jnp.int32, (1, _MAX_PEAKS), 1)
    px_row = jnp.zeros((1, _MAX_PEAKS), jnp.float32)
    py_row = jnp.zeros((1, _MAX_PEAKS), jnp.float32)
    scr_row = jnp.zeros((1, _MAX_PEAKS), jnp.float32)
    val_row = jnp.zeros((1, _MAX_PEAKS), jnp.float32)

    for n in range(_MAX_PEAKS):
        m = jnp.max(masked)
        idx = jnp.min(jnp.where(masked == m, lin, jnp.int32(1 << 30)))
        y = idx // _W
        x = idx % _W
        xp = jnp.minimum(x + 1, _W - 1)
        xm = jnp.maximum(x - 1, 0)
        yp = jnp.minimum(y + 1, _H - 1)
        ym = jnp.maximum(y - 1, 0)
        row0 = h_ref[0, pl.ds(y, 1), :]
        rowp = h_ref[0, pl.ds(yp, 1), :]
        rowm = h_ref[0, pl.ds(ym, 1), :]
        ci1 = lax.broadcasted_iota(jnp.int32, (1, _W), 1)

        def pick(row, xx):
            return jnp.sum(jnp.where(ci1 == xx, row, 0.0))

        v0 = pick(row0, x)
        vxp = pick(row0, xp)
        vxm = pick(row0, xm)
        vyp = pick(rowp, x)
        vym = pick(rowm, x)
        dx_raw = 0.5 * (vxp - vxm)
        dy_raw = 0.5 * (vyp - vym)
        dxx = vxp + vxm - 2.0 * v0
        dyy = vyp + vym - 2.0 * v0
        gx = jnp.abs(dxx) > 1e-6
        gy = jnp.abs(dyy) > 1e-6
        dx = jnp.where(gx, dx_raw / -jnp.where(gx, dxx, 1.0), dx_raw)
        dy = jnp.where(gy, dy_raw / -jnp.where(gy, dyy, 1.0), dy_raw)
        interior = (x > 0) & (x < _W - 1) & (y > 0) & (y < _H - 1)
        pxv = x.astype(jnp.float32) + jnp.where(interior, dx, 0.0)
        pyv = y.astype(jnp.float32) + jnp.where(interior, dy, 0.0)
        validn = m > _PEAK_THRESH
        hit = li20 == n
        px_row = jnp.where(hit, pxv, px_row)
        py_row = jnp.where(hit, pyv, py_row)
        scr_row = jnp.where(hit, jnp.where(validn, m, 0.0), scr_row)
        val_row = jnp.where(hit, jnp.where(validn, 1.0, 0.0), val_row)
        masked = jnp.where(lin == idx, jnp.float32(-2e9), masked)

    px_ref[0] = px_row
    py_ref[0] = py_row
    scr_ref[0] = scr_row
    val_ref[0] = val_row


def _geom_body(pxa_ref, pya_ref, sa_ref, va_ref, pxb_ref, pyb_ref, sb_ref,
               vb_ref, t_ref, lin_ref, ux_ref, uy_ref, vm_ref, at_ref):
    rows = pxa_ref.shape[0]
    shp = (rows, _MAX_PEAKS, _MAX_PEAKS)
    ax = jnp.broadcast_to(pxa_ref[...][:, :, None], shp)
    ay = jnp.broadcast_to(pya_ref[...][:, :, None], shp)
    bx = jnp.broadcast_to(pxb_ref[...][:, None, :], shp)
    by = jnp.broadcast_to(pyb_ref[...][:, None, :], shp)
    ddx = bx - ax
    ddy = by - ay
    norm = jnp.sqrt(ddx * ddx + ddy * ddy + 1e-12) + 1e-8
    ux_ref[...] = ddx / norm
    uy_ref[...] = ddy / norm
    va = jnp.broadcast_to(va_ref[...][:, :, None], shp)
    vb = jnp.broadcast_to(vb_ref[...][:, None, :], shp)
    vm_ref[...] = va * vb
    sa = jnp.broadcast_to(sa_ref[...][:, :, None], shp)
    sb = jnp.broadcast_to(sb_ref[...][:, None, :], shp)
    at_ref[...] = 0.5 * (sa + sb)
    for s in range(_N_SAMPLES):
        ts = t_ref[0, 0, s]
        xl = ax + ddx * ts
        yl = ay + ddy * ts
        ix = jnp.clip(jnp.round(xl).astype(jnp.int32), 0, _W - 1)
        iy = jnp.clip(jnp.round(yl).astype(jnp.int32), 0, _H - 1)
        lin_ref[:, s] = iy * _W + ix


def _sc_body(pafx_hbm, pafy_hbm, lin_hbm, ux_hbm, uy_hbm, vm_hbm, at_hbm,
             out_hbm, pafx_v, pafy_v, lin_v, ux_v, uy_v, vm_v, at_v, acc_v):
    n_items = lin_hbm.shape[0]
    wid = lax.axis_index("s") * 2 + lax.axis_index("c")
    n_workers = 32
    n_iters = (n_items + n_workers - 1) // n_workers
    n_chunks = (_MAX_PEAKS * _MAX_PEAKS) // 16
    for it in range(n_iters):
        bc = wid + n_workers * it

        @pl.when(bc < n_items)
        def _():
            pltpu.sync_copy(pafx_hbm.at[bc], pafx_v)
            pltpu.sync_copy(pafy_hbm.at[bc], pafy_v)
            pltpu.sync_copy(lin_hbm.at[bc], lin_v)
            pltpu.sync_copy(ux_hbm.at[bc], ux_v)
            pltpu.sync_copy(uy_hbm.at[bc], uy_v)
            pltpu.sync_copy(vm_hbm.at[bc], vm_v)
            pltpu.sync_copy(at_hbm.at[bc], at_v)

            def pbody(p, carry):
                off = pl.multiple_of(p * 16, 16)
                uxv = ux_v[pl.ds(off, 16)]
                uyv = uy_v[pl.ds(off, 16)]
                vmv = vm_v[pl.ds(off, 16)]
                atv = at_v[pl.ds(off, 16)]
                acc = jnp.zeros((16,), jnp.float32)
                cnt = jnp.zeros((16,), jnp.float32)
                for s in range(_N_SAMPLES):
                    linv = lin_v[s, pl.ds(off, 16)]
                    sx = plsc.load_gather(pafx_v, [linv])
                    sy = plsc.load_gather(pafy_v, [linv])
                    vec = sx * uxv + sy * uyv
                    acc = acc + vec
                    cnt = cnt + jnp.where(
                        vec > _PAF_SCORE_THRESH,
                        jnp.float32(1.0), jnp.float32(0.0))
                mean = acc / jnp.float32(_N_SAMPLES)
                ok = (mean > 0.0) & (cnt > 8.0) & (vmv > 0.5)
                acc_v[pl.ds(off, 16)] = jnp.where(ok, mean + atv, 0.0)
                return carry

            lax.fori_loop(0, n_chunks, pbody, 0)
            pltpu.sync_copy(acc_v, out_hbm.at[bc])


def _run_peaks(h3):
    n = h3.shape[0]
    out = jax.ShapeDtypeStruct((n, 1, _MAX_PEAKS), jnp.float32)
    return pl.pallas_call(
        _peaks_body,
        grid=(n,),
        in_specs=[pl.BlockSpec((1, _H, _W), lambda i: (i, 0, 0))],
        out_specs=[pl.BlockSpec((1, 1, _MAX_PEAKS), lambda i: (i, 0, 0))] * 4,
        out_shape=[out] * 4,
    )(h3)


def _run_geom(pxa, pya, sa, va, pxb, pyb, sb, vb, t16):
    n = pxa.shape[0]
    rows = 8
    grid = (n // rows,)
    vec_spec = pl.BlockSpec((rows, _MAX_PEAKS), lambda i: (i, 0))
    mat_spec = pl.BlockSpec((rows, _MAX_PEAKS, _MAX_PEAKS),
                            lambda i: (i, 0, 0))
    lin_spec = pl.BlockSpec((rows, _N_SAMPLES, _MAX_PEAKS, _MAX_PEAKS),
                            lambda i: (i, 0, 0, 0))
    return pl.pallas_call(
        _geom_body,
        grid=grid,
        in_specs=[vec_spec] * 8 + [
            pl.BlockSpec((1, 1, 16), lambda i: (0, 0, 0))],
        out_specs=[lin_spec, mat_spec, mat_spec, mat_spec, mat_spec],
        out_shape=[
            jax.ShapeDtypeStruct(
                (n, _N_SAMPLES, _MAX_PEAKS, _MAX_PEAKS), jnp.int32),
            jax.ShapeDtypeStruct((n, _MAX_PEAKS, _MAX_PEAKS), jnp.float32),
            jax.ShapeDtypeStruct((n, _MAX_PEAKS, _MAX_PEAKS), jnp.float32),
            jax.ShapeDtypeStruct((n, _MAX_PEAKS, _MAX_PEAKS), jnp.float32),
            jax.ShapeDtypeStruct((n, _MAX_PEAKS, _MAX_PEAKS), jnp.float32),
        ],
    )(pxa, pya, sa, va, pxb, pyb, sb, vb, t16)


def _run_sc(pafx, pafy, lin, ux, uy, vm, at):
    n = lin.shape[0]
    npair = _MAX_PEAKS * _MAX_PEAKS
    mesh = plsc.VectorSubcoreMesh(core_axis_name="c", subcore_axis_name="s")
    fn = functools.partial(
        pl.kernel,
        out_type=jax.ShapeDtypeStruct((n, npair), jnp.float32),
        mesh=mesh,
        compiler_params=pltpu.CompilerParams(needs_layout_passes=False),
        scratch_types=[
            pltpu.VMEM((_H * _W,), jnp.float32),
            pltpu.VMEM((_H * _W,), jnp.float32),
            pltpu.VMEM((_N_SAMPLES, npair), jnp.int32),
            pltpu.VMEM((npair,), jnp.float32),
            pltpu.VMEM((npair,), jnp.float32),
            pltpu.VMEM((npair,), jnp.float32),
            pltpu.VMEM((npair,), jnp.float32),
            pltpu.VMEM((npair,), jnp.float32),
        ],
    )(_sc_body)
    return fn(pafx, pafy, lin, ux, uy, vm, at)


def kernel(heat_pred, paf_pred):
    B, K, H, W = heat_pred.shape
    h3 = heat_pred.reshape(B * K, H, W)
    px3, py3, scr3, val3 = _run_peaks(h3)
    px = px3.reshape(B, K, _MAX_PEAKS)
    py = py3.reshape(B, K, _MAX_PEAKS)
    scr = scr3.reshape(B, K, _MAX_PEAKS)
    valf = val3.reshape(B, K, _MAX_PEAKS)

    a_idx = jnp.asarray(_SKEL[:, 0])
    b_idx = jnp.asarray(_SKEL[:, 1])
    C = _SKEL.shape[0]
    n = B * C
    pxa = px[:, a_idx].reshape(n, _MAX_PEAKS)
    pya = py[:, a_idx].reshape(n, _MAX_PEAKS)
    sa = scr[:, a_idx].reshape(n, _MAX_PEAKS)
    va = valf[:, a_idx].reshape(n, _MAX_PEAKS)
    pxb = px[:, b_idx].reshape(n, _MAX_PEAKS)
    pyb = py[:, b_idx].reshape(n, _MAX_PEAKS)
    sb = scr[:, b_idx].reshape(n, _MAX_PEAKS)
    vb = valf[:, b_idx].reshape(n, _MAX_PEAKS)

    t = jnp.linspace(0.0, 1.0, _N_SAMPLES).astype(jnp.float32)
    t16 = jnp.zeros((1, 1, 16), jnp.float32).at[0, 0, :_N_SAMPLES].set(t)

    lin, ux, uy, vm, at = _run_geom(pxa, pya, sa, va, pxb, pyb, sb, vb, t16)
    npair = _MAX_PEAKS * _MAX_PEAKS
    lin = lin.reshape(n, _N_SAMPLES, npair)
    ux = ux.reshape(n, npair)
    uy = uy.reshape(n, npair)
    vm = vm.reshape(n, npair)
    at = at.reshape(n, npair)

    pafx = paf_pred[:, 0::2].reshape(n, H * W)
    pafy = paf_pred[:, 1::2].reshape(n, H * W)

    conn = _run_sc(pafx, pafy, lin, ux, uy, vm, at)
    conn_scores = conn.reshape(B, C, _MAX_PEAKS, _MAX_PEAKS)
    return (px, py, scr, conn_scores)


# speedup vs baseline: 6.3661x; 3.6095x over previous
import functools

import jax
import jax.numpy as jnp
import numpy as np
from jax import lax
from jax.experimental import pallas as pl
from jax.experimental.pallas import tpu as pltpu
from jax.experimental.pallas import tpu_sc as plsc

_SKEL = np.array(
    [[15, 13], [13, 11], [16, 14], [14, 12], [11, 12], [5, 11], [6, 12],
     [5, 6], [5, 7], [6, 8], [7, 9], [8, 10], [1, 2], [0, 1], [0, 2],
     [1, 3], [2, 4], [3, 5], [4, 6]], dtype=np.int32)

_PEAK_THRESH = 0.1
_PAF_SCORE_THRESH = 0.05
_MAX_PEAKS = 20
_N_SAMPLES = 10
_NEG = -1e9
_H = 128
_W = 128
_NINF = float("-inf")


_G = 8


def _peaks_body(h_ref, px_ref, py_ref, scr_ref, val_ref, msk_ref):
    h = h_ref[...]
    col_ninf = jnp.full((_G, _H, 1), _NINF, jnp.float32)
    s_l = jnp.concatenate([h[:, :, 1:], col_ninf], axis=2)
    s_r = jnp.concatenate([col_ninf, h[:, :, :-1]], axis=2)
    rmax = jnp.maximum(h, jnp.maximum(s_l, s_r))
    row_ninf = jnp.full((_G, 1, _W), _NINF, jnp.float32)
    s_u = jnp.concatenate([rmax[:, 1:, :], row_ninf], axis=1)
    s_d = jnp.concatenate([row_ninf, rmax[:, :-1, :]], axis=1)
    pooled = jnp.maximum(rmax, jnp.maximum(s_u, s_d))

    is_peak = (h == pooled) & (h > _PEAK_THRESH)
    masked = jnp.where(is_peak, h, jnp.float32(_NEG))
    msk_ref[...] = masked
    rm0 = jnp.max(masked, axis=2)

    siG = lax.broadcasted_iota(jnp.int32, (_G, _H), 1)
    ciG = lax.broadcasted_iota(jnp.int32, (_G, _W), 1)
    li20 = lax.broadcasted_iota(jnp.int32, (_G, _MAX_PEAKS), 1)
    zmat = jnp.zeros((_G, _MAX_PEAKS), jnp.float32)
    big = jnp.int32(1 << 30)

    def body(n, carry):
        rm, px_m, py_m, scr_m, val_m = carry
        m = jnp.max(rm, axis=1, keepdims=True)
        ysel = jnp.min(jnp.where(rm == m, siG, big), axis=1, keepdims=True)
        rows = []
        rows_p = []
        rows_m = []
        for i in range(_G):
            yv = ysel[i, 0]
            rows.append(msk_ref[i, pl.ds(yv, 1), :])
            rows_p.append(h_ref[i, pl.ds(jnp.minimum(yv + 1, _H - 1), 1), :])
            rows_m.append(h_ref[i, pl.ds(jnp.maximum(yv - 1, 0), 1), :])
        R = jnp.concatenate(rows, axis=0)
        Hp = jnp.concatenate(rows_p, axis=0)
        Hm = jnp.concatenate(rows_m, axis=0)
        xsel = jnp.min(jnp.where(R == m, ciG, big), axis=1, keepdims=True)
        newR = jnp.where(ciG == xsel, jnp.float32(-2e9), R)
        for i in range(_G):
            msk_ref[i, pl.ds(ysel[i, 0], 1), :] = newR[i:i + 1, :]
        rm = jnp.where(siG == ysel, jnp.max(newR, axis=1, keepdims=True), rm)

        rows_0 = []
        for i in range(_G):
            rows_0.append(h_ref[i, pl.ds(ysel[i, 0], 1), :])
        H0 = jnp.concatenate(rows_0, axis=0)

        def pick(rowmat, xx):
            return jnp.sum(jnp.where(ciG == xx, rowmat, 0.0),
                           axis=1, keepdims=True)

        xp = jnp.minimum(xsel + 1, _W - 1)
        xm = jnp.maximum(xsel - 1, 0)
        v0 = pick(H0, xsel)
        vxp = pick(H0, xp)
        vxm = pick(H0, xm)
        vyp = pick(Hp, xsel)
        vym = pick(Hm, xsel)
        dx_raw = 0.5 * (vxp - vxm)
        dy_raw = 0.5 * (vyp - vym)
        dxx = vxp + vxm - 2.0 * v0
        dyy = vyp + vym - 2.0 * v0
        gx = jnp.abs(dxx) > 1e-6
        gy = jnp.abs(dyy) > 1e-6
        dx = jnp.where(gx, dx_raw / -jnp.where(gx, dxx, 1.0), dx_raw)
        dy = jnp.where(gy, dy_raw / -jnp.where(gy, dyy, 1.0), dy_raw)
        interior = (xsel > 0) & (xsel < _W - 1) & (ysel > 0) & (ysel < _H - 1)
        pxv = xsel.astype(jnp.float32) + jnp.where(interior, dx, 0.0)
        pyv = ysel.astype(jnp.float32) + jnp.where(interior, dy, 0.0)
        validn = m > _PEAK_THRESH
        hit = li20 == n
        px_m = jnp.where(hit, pxv, px_m)
        py_m = jnp.where(hit, pyv, py_m)
        scr_m = jnp.where(hit, jnp.where(validn, m, 0.0), scr_m)
        val_m = jnp.where(hit, jnp.where(validn, 1.0, 0.0), val_m)
        return rm, px_m, py_m, scr_m, val_m

    _, px_m, py_m, scr_m, val_m = lax.fori_loop(
        0, _MAX_PEAKS, body, (rm0, zmat, zmat, zmat, zmat))
    px_ref[...] = px_m
    py_ref[...] = py_m
    scr_ref[...] = scr_m
    val_ref[...] = val_m


def _geom_body(pxa_ref, pya_ref, sa_ref, va_ref, pxb_ref, pyb_ref, sb_ref,
               vb_ref, t_ref, lin_ref, ux_ref, uy_ref, vm_ref, at_ref):
    rows = pxa_ref.shape[0]
    shp = (rows, _MAX_PEAKS, _MAX_PEAKS)
    ax = jnp.broadcast_to(pxa_ref[...][:, :, None], shp)
    ay = jnp.broadcast_to(pya_ref[...][:, :, None], shp)
    bx = jnp.broadcast_to(pxb_ref[...][:, None, :], shp)
    by = jnp.broadcast_to(pyb_ref[...][:, None, :], shp)
    ddx = bx - ax
    ddy = by - ay
    norm = jnp.sqrt(ddx * ddx + ddy * ddy + 1e-12) + 1e-8
    ux_ref[...] = ddx / norm
    uy_ref[...] = ddy / norm
    va = jnp.broadcast_to(va_ref[...][:, :, None], shp)
    vb = jnp.broadcast_to(vb_ref[...][:, None, :], shp)
    vm_ref[...] = va * vb
    sa = jnp.broadcast_to(sa_ref[...][:, :, None], shp)
    sb = jnp.broadcast_to(sb_ref[...][:, None, :], shp)
    at_ref[...] = 0.5 * (sa + sb)
    for s in range(_N_SAMPLES):
        ts = t_ref[0, 0, s]
        xl = ax + ddx * ts
        yl = ay + ddy * ts
        ix = jnp.clip(jnp.round(xl).astype(jnp.int32), 0, _W - 1)
        iy = jnp.clip(jnp.round(yl).astype(jnp.int32), 0, _H - 1)
        lin_ref[:, s] = iy * _W + ix


def _sc_body(pafx_hbm, pafy_hbm, lin_hbm, ux_hbm, uy_hbm, vm_hbm, at_hbm,
             out_hbm, pafx_v, pafy_v, lin_v, ux_v, uy_v, vm_v, at_v, acc_v):
    n_items = lin_hbm.shape[0]
    wid = lax.axis_index("s") * 2 + lax.axis_index("c")
    n_workers = 32
    n_iters = (n_items + n_workers - 1) // n_workers
    n_chunks = (_MAX_PEAKS * _MAX_PEAKS) // 16
    for it in range(n_iters):
        bc = wid + n_workers * it

        @pl.when(bc < n_items)
        def _():
            pltpu.sync_copy(pafx_hbm.at[bc], pafx_v)
            pltpu.sync_copy(pafy_hbm.at[bc], pafy_v)
            pltpu.sync_copy(lin_hbm.at[bc], lin_v)
            pltpu.sync_copy(ux_hbm.at[bc], ux_v)
            pltpu.sync_copy(uy_hbm.at[bc], uy_v)
            pltpu.sync_copy(vm_hbm.at[bc], vm_v)
            pltpu.sync_copy(at_hbm.at[bc], at_v)

            def pbody(p, carry):
                off = pl.multiple_of(p * 16, 16)
                uxv = ux_v[pl.ds(off, 16)]
                uyv = uy_v[pl.ds(off, 16)]
                vmv = vm_v[pl.ds(off, 16)]
                atv = at_v[pl.ds(off, 16)]
                acc = jnp.zeros((16,), jnp.float32)
                cnt = jnp.zeros((16,), jnp.float32)
                for s in range(_N_SAMPLES):
                    linv = lin_v[s, pl.ds(off, 16)]
                    sx = plsc.load_gather(pafx_v, [linv])
                    sy = plsc.load_gather(pafy_v, [linv])
                    vec = sx * uxv + sy * uyv
                    acc = acc + vec
                    cnt = cnt + jnp.where(
                        vec > _PAF_SCORE_THRESH,
                        jnp.float32(1.0), jnp.float32(0.0))
                mean = acc / jnp.float32(_N_SAMPLES)
                ok = (mean > 0.0) & (cnt > 8.0) & (vmv > 0.5)
                acc_v[pl.ds(off, 16)] = jnp.where(ok, mean + atv, 0.0)
                return carry

            lax.fori_loop(0, n_chunks, pbody, 0)
            pltpu.sync_copy(acc_v, out_hbm.at[bc])


def _run_peaks(h3):
    n = h3.shape[0]
    out = jax.ShapeDtypeStruct((n, _MAX_PEAKS), jnp.float32)
    return pl.pallas_call(
        _peaks_body,
        grid=(n // _G,),
        in_specs=[pl.BlockSpec((_G, _H, _W), lambda i: (i, 0, 0))],
        out_specs=[pl.BlockSpec((_G, _MAX_PEAKS), lambda i: (i, 0))] * 4,
        out_shape=[out] * 4,
        scratch_shapes=[pltpu.VMEM((_G, _H, _W), jnp.float32)],
    )(h3)


def _run_geom(pxa, pya, sa, va, pxb, pyb, sb, vb, t16):
    n = pxa.shape[0]
    rows = 8
    grid = (n // rows,)
    vec_spec = pl.BlockSpec((rows, _MAX_PEAKS), lambda i: (i, 0))
    mat_spec = pl.BlockSpec((rows, _MAX_PEAKS, _MAX_PEAKS),
                            lambda i: (i, 0, 0))
    lin_spec = pl.BlockSpec((rows, _N_SAMPLES, _MAX_PEAKS, _MAX_PEAKS),
                            lambda i: (i, 0, 0, 0))
    return pl.pallas_call(
        _geom_body,
        grid=grid,
        in_specs=[vec_spec] * 8 + [
            pl.BlockSpec((1, 1, 16), lambda i: (0, 0, 0))],
        out_specs=[lin_spec, mat_spec, mat_spec, mat_spec, mat_spec],
        out_shape=[
            jax.ShapeDtypeStruct(
                (n, _N_SAMPLES, _MAX_PEAKS, _MAX_PEAKS), jnp.int32),
            jax.ShapeDtypeStruct((n, _MAX_PEAKS, _MAX_PEAKS), jnp.float32),
            jax.ShapeDtypeStruct((n, _MAX_PEAKS, _MAX_PEAKS), jnp.float32),
            jax.ShapeDtypeStruct((n, _MAX_PEAKS, _MAX_PEAKS), jnp.float32),
            jax.ShapeDtypeStruct((n, _MAX_PEAKS, _MAX_PEAKS), jnp.float32),
        ],
    )(pxa, pya, sa, va, pxb, pyb, sb, vb, t16)


def _run_sc(pafx, pafy, lin, ux, uy, vm, at):
    n = lin.shape[0]
    npair = _MAX_PEAKS * _MAX_PEAKS
    mesh = plsc.VectorSubcoreMesh(core_axis_name="c", subcore_axis_name="s")
    fn = functools.partial(
        pl.kernel,
        out_type=jax.ShapeDtypeStruct((n, npair), jnp.float32),
        mesh=mesh,
        compiler_params=pltpu.CompilerParams(needs_layout_passes=False),
        scratch_types=[
            pltpu.VMEM((_H * _W,), jnp.float32),
            pltpu.VMEM((_H * _W,), jnp.float32),
            pltpu.VMEM((_N_SAMPLES, npair), jnp.int32),
            pltpu.VMEM((npair,), jnp.float32),
            pltpu.VMEM((npair,), jnp.float32),
            pltpu.VMEM((npair,), jnp.float32),
            pltpu.VMEM((npair,), jnp.float32),
            pltpu.VMEM((npair,), jnp.float32),
        ],
    )(_sc_body)
    return fn(pafx, pafy, lin, ux, uy, vm, at)


def kernel(heat_pred, paf_pred):
    B, K, H, W = heat_pred.shape
    h3 = heat_pred.reshape(B * K, H, W)
    px3, py3, scr3, val3 = _run_peaks(h3)
    px = px3.reshape(B, K, _MAX_PEAKS)
    py = py3.reshape(B, K, _MAX_PEAKS)
    scr = scr3.reshape(B, K, _MAX_PEAKS)
    valf = val3.reshape(B, K, _MAX_PEAKS)

    a_idx = jnp.asarray(_SKEL[:, 0])
    b_idx = jnp.asarray(_SKEL[:, 1])
    C = _SKEL.shape[0]
    n = B * C
    pxa = px[:, a_idx].reshape(n, _MAX_PEAKS)
    pya = py[:, a_idx].reshape(n, _MAX_PEAKS)
    sa = scr[:, a_idx].reshape(n, _MAX_PEAKS)
    va = valf[:, a_idx].reshape(n, _MAX_PEAKS)
    pxb = px[:, b_idx].reshape(n, _MAX_PEAKS)
    pyb = py[:, b_idx].reshape(n, _MAX_PEAKS)
    sb = scr[:, b_idx].reshape(n, _MAX_PEAKS)
    vb = valf[:, b_idx].reshape(n, _MAX_PEAKS)

    t = jnp.linspace(0.0, 1.0, _N_SAMPLES).astype(jnp.float32)
    t16 = jnp.zeros((1, 1, 16), jnp.float32).at[0, 0, :_N_SAMPLES].set(t)

    lin, ux, uy, vm, at = _run_geom(pxa, pya, sa, va, pxb, pyb, sb, vb, t16)
    npair = _MAX_PEAKS * _MAX_PEAKS
    lin = lin.reshape(n, _N_SAMPLES, npair)
    ux = ux.reshape(n, npair)
    uy = uy.reshape(n, npair)
    vm = vm.reshape(n, npair)
    at = at.reshape(n, npair)

    pafx = paf_pred[:, 0::2].reshape(n, H * W)
    pafy = paf_pred[:, 1::2].reshape(n, H * W)

    conn = _run_sc(pafx, pafy, lin, ux, uy, vm, at)
    conn_scores = conn.reshape(B, C, _MAX_PEAKS, _MAX_PEAKS)
    return (px, py, scr, conn_scores)


# speedup vs baseline: 7.0568x; 1.1085x over previous
import functools

import jax
import jax.numpy as jnp
import numpy as np
from jax import lax
from jax.experimental import pallas as pl
from jax.experimental.pallas import tpu as pltpu
from jax.experimental.pallas import tpu_sc as plsc

_SKEL = np.array(
    [[15, 13], [13, 11], [16, 14], [14, 12], [11, 12], [5, 11], [6, 12],
     [5, 6], [5, 7], [6, 8], [7, 9], [8, 10], [1, 2], [0, 1], [0, 2],
     [1, 3], [2, 4], [3, 5], [4, 6]], dtype=np.int32)

_PEAK_THRESH = 0.1
_PAF_SCORE_THRESH = 0.05
_MAX_PEAKS = 20
_N_SAMPLES = 10
_NEG = -1e9
_H = 128
_W = 128
_NINF = float("-inf")


_G = 8


def _peaks_body(h_ref, px_ref, py_ref, scr_ref, val_ref, msk_ref):
    h = h_ref[...]
    col_ninf = jnp.full((_G, _H, 1), _NINF, jnp.float32)
    s_l = jnp.concatenate([h[:, :, 1:], col_ninf], axis=2)
    s_r = jnp.concatenate([col_ninf, h[:, :, :-1]], axis=2)
    rmax = jnp.maximum(h, jnp.maximum(s_l, s_r))
    row_ninf = jnp.full((_G, 1, _W), _NINF, jnp.float32)
    s_u = jnp.concatenate([rmax[:, 1:, :], row_ninf], axis=1)
    s_d = jnp.concatenate([row_ninf, rmax[:, :-1, :]], axis=1)
    pooled = jnp.maximum(rmax, jnp.maximum(s_u, s_d))

    is_peak = (h == pooled) & (h > _PEAK_THRESH)
    masked = jnp.where(is_peak, h, jnp.float32(_NEG))
    msk_ref[...] = masked
    rm0 = jnp.max(masked, axis=2)

    siG = lax.broadcasted_iota(jnp.int32, (_G, _H), 1)
    ciG = lax.broadcasted_iota(jnp.int32, (_G, _W), 1)
    li20 = lax.broadcasted_iota(jnp.int32, (_G, _MAX_PEAKS), 1)
    zmat = jnp.zeros((_G, _MAX_PEAKS), jnp.float32)
    big = jnp.int32(1 << 30)

    def body(n, carry):
        rm, px_m, py_m, scr_m, val_m = carry
        m = jnp.max(rm, axis=1, keepdims=True)
        ysel = jnp.min(jnp.where(rm == m, siG, big), axis=1, keepdims=True)
        rows = []
        rows_p = []
        rows_m = []
        for i in range(_G):
            yv = ysel[i, 0]
            rows.append(msk_ref[i, pl.ds(yv, 1), :])
            rows_p.append(h_ref[i, pl.ds(jnp.minimum(yv + 1, _H - 1), 1), :])
            rows_m.append(h_ref[i, pl.ds(jnp.maximum(yv - 1, 0), 1), :])
        R = jnp.concatenate(rows, axis=0)
        Hp = jnp.concatenate(rows_p, axis=0)
        Hm = jnp.concatenate(rows_m, axis=0)
        xsel = jnp.min(jnp.where(R == m, ciG, big), axis=1, keepdims=True)
        newR = jnp.where(ciG == xsel, jnp.float32(-2e9), R)
        for i in range(_G):
            msk_ref[i, pl.ds(ysel[i, 0], 1), :] = newR[i:i + 1, :]
        rm = jnp.where(siG == ysel, jnp.max(newR, axis=1, keepdims=True), rm)

        rows_0 = []
        for i in range(_G):
            rows_0.append(h_ref[i, pl.ds(ysel[i, 0], 1), :])
        H0 = jnp.concatenate(rows_0, axis=0)

        def pick(rowmat, xx):
            return jnp.sum(jnp.where(ciG == xx, rowmat, 0.0),
                           axis=1, keepdims=True)

        xp = jnp.minimum(xsel + 1, _W - 1)
        xm = jnp.maximum(xsel - 1, 0)
        v0 = pick(H0, xsel)
        vxp = pick(H0, xp)
        vxm = pick(H0, xm)
        vyp = pick(Hp, xsel)
        vym = pick(Hm, xsel)
        dx_raw = 0.5 * (vxp - vxm)
        dy_raw = 0.5 * (vyp - vym)
        dxx = vxp + vxm - 2.0 * v0
        dyy = vyp + vym - 2.0 * v0
        gx = jnp.abs(dxx) > 1e-6
        gy = jnp.abs(dyy) > 1e-6
        dx = jnp.where(gx, dx_raw / -jnp.where(gx, dxx, 1.0), dx_raw)
        dy = jnp.where(gy, dy_raw / -jnp.where(gy, dyy, 1.0), dy_raw)
        interior = (xsel > 0) & (xsel < _W - 1) & (ysel > 0) & (ysel < _H - 1)
        pxv = xsel.astype(jnp.float32) + jnp.where(interior, dx, 0.0)
        pyv = ysel.astype(jnp.float32) + jnp.where(interior, dy, 0.0)
        validn = m > _PEAK_THRESH
        hit = li20 == n
        px_m = jnp.where(hit, pxv, px_m)
        py_m = jnp.where(hit, pyv, py_m)
        scr_m = jnp.where(hit, jnp.where(validn, m, 0.0), scr_m)
        val_m = jnp.where(hit, jnp.where(validn, 1.0, 0.0), val_m)
        return rm, px_m, py_m, scr_m, val_m

    _, px_m, py_m, scr_m, val_m = lax.fori_loop(
        0, _MAX_PEAKS, body, (rm0, zmat, zmat, zmat, zmat))
    px_ref[...] = px_m
    py_ref[...] = py_m
    scr_ref[...] = scr_m
    val_ref[...] = val_m


def _geom_body(pxa_ref, pya_ref, sa_ref, va_ref, pxb_ref, pyb_ref, sb_ref,
               vb_ref, t_ref, lin_ref, ux_ref, uy_ref, vm_ref, at_ref):
    rows = pxa_ref.shape[0]
    shp = (rows, _MAX_PEAKS, _MAX_PEAKS)
    ax = jnp.broadcast_to(pxa_ref[...][:, :, None], shp)
    ay = jnp.broadcast_to(pya_ref[...][:, :, None], shp)
    bx = jnp.broadcast_to(pxb_ref[...][:, None, :], shp)
    by = jnp.broadcast_to(pyb_ref[...][:, None, :], shp)
    ddx = bx - ax
    ddy = by - ay
    norm = jnp.sqrt(ddx * ddx + ddy * ddy + 1e-12) + 1e-8
    ux_ref[...] = ddx / norm
    uy_ref[...] = ddy / norm
    va = jnp.broadcast_to(va_ref[...][:, :, None], shp)
    vb = jnp.broadcast_to(vb_ref[...][:, None, :], shp)
    vm_ref[...] = va * vb
    sa = jnp.broadcast_to(sa_ref[...][:, :, None], shp)
    sb = jnp.broadcast_to(sb_ref[...][:, None, :], shp)
    at_ref[...] = 0.5 * (sa + sb)
    for s in range(_N_SAMPLES):
        ts = t_ref[0, 0, s]
        xl = ax + ddx * ts
        yl = ay + ddy * ts
        ix = jnp.clip(jnp.round(xl).astype(jnp.int32), 0, _W - 1)
        iy = jnp.clip(jnp.round(yl).astype(jnp.int32), 0, _H - 1)
        lin_ref[:, s] = iy * _W + ix


def _sc_body(paf_hbm, lin_hbm, ux_hbm, uy_hbm, vm_hbm, at_hbm,
             out_hbm, pafx_v, pafy_v, lin_v, ux_v, uy_v, vm_v, at_v, acc_v):
    n_items = lin_hbm.shape[0]
    n_conn = _SKEL.shape[0]
    wid = lax.axis_index("s") * 2 + lax.axis_index("c")
    n_workers = 32
    n_iters = (n_items + n_workers - 1) // n_workers
    n_chunks = (_MAX_PEAKS * _MAX_PEAKS) // 16
    for it in range(n_iters):
        bc = wid + n_workers * it

        @pl.when(bc < n_items)
        def _():
            pltpu.sync_copy(paf_hbm.at[2 * bc], pafx_v)
            pltpu.sync_copy(paf_hbm.at[2 * bc + 1], pafy_v)
            pltpu.sync_copy(lin_hbm.at[bc], lin_v)
            pltpu.sync_copy(ux_hbm.at[bc], ux_v)
            pltpu.sync_copy(uy_hbm.at[bc], uy_v)
            pltpu.sync_copy(vm_hbm.at[bc], vm_v)
            pltpu.sync_copy(at_hbm.at[bc], at_v)

            def pbody(p, carry):
                off = pl.multiple_of(p * 16, 16)
                uxv = ux_v[pl.ds(off, 16)]
                uyv = uy_v[pl.ds(off, 16)]
                vmv = vm_v[pl.ds(off, 16)]
                atv = at_v[pl.ds(off, 16)]
                acc = jnp.zeros((16,), jnp.float32)
                cnt = jnp.zeros((16,), jnp.float32)
                for s in range(_N_SAMPLES):
                    linv = lin_v[s, pl.ds(off, 16)]
                    sx = plsc.load_gather(pafx_v, [linv])
                    sy = plsc.load_gather(pafy_v, [linv])
                    vec = sx * uxv + sy * uyv
                    acc = acc + vec
                    cnt = cnt + jnp.where(
                        vec > _PAF_SCORE_THRESH,
                        jnp.float32(1.0), jnp.float32(0.0))
                mean = acc / jnp.float32(_N_SAMPLES)
                ok = (mean > 0.0) & (cnt > 8.0) & (vmv > 0.5)
                acc_v[pl.ds(off, 16)] = jnp.where(ok, mean + atv, 0.0)
                return carry

            lax.fori_loop(0, n_chunks, pbody, 0)
            pltpu.sync_copy(acc_v, out_hbm.at[bc])


def _run_peaks(h3):
    n = h3.shape[0]
    out = jax.ShapeDtypeStruct((n, _MAX_PEAKS), jnp.float32)
    return pl.pallas_call(
        _peaks_body,
        grid=(n // _G,),
        in_specs=[pl.BlockSpec((_G, _H, _W), lambda i: (i, 0, 0))],
        out_specs=[pl.BlockSpec((_G, _MAX_PEAKS), lambda i: (i, 0))] * 4,
        out_shape=[out] * 4,
        scratch_shapes=[pltpu.VMEM((_G, _H, _W), jnp.float32)],
    )(h3)


def _run_geom(pxa, pya, sa, va, pxb, pyb, sb, vb, t16):
    n = pxa.shape[0]
    rows = 8
    grid = (n // rows,)
    vec_spec = pl.BlockSpec((rows, _MAX_PEAKS), lambda i: (i, 0))
    mat_spec = pl.BlockSpec((rows, _MAX_PEAKS, _MAX_PEAKS),
                            lambda i: (i, 0, 0))
    lin_spec = pl.BlockSpec((rows, _N_SAMPLES, _MAX_PEAKS, _MAX_PEAKS),
                            lambda i: (i, 0, 0, 0))
    return pl.pallas_call(
        _geom_body,
        grid=grid,
        in_specs=[vec_spec] * 8 + [
            pl.BlockSpec((1, 1, 16), lambda i: (0, 0, 0))],
        out_specs=[lin_spec, mat_spec, mat_spec, mat_spec, mat_spec],
        out_shape=[
            jax.ShapeDtypeStruct(
                (n, _N_SAMPLES, _MAX_PEAKS, _MAX_PEAKS), jnp.int32),
            jax.ShapeDtypeStruct((n, _MAX_PEAKS, _MAX_PEAKS), jnp.float32),
            jax.ShapeDtypeStruct((n, _MAX_PEAKS, _MAX_PEAKS), jnp.float32),
            jax.ShapeDtypeStruct((n, _MAX_PEAKS, _MAX_PEAKS), jnp.float32),
            jax.ShapeDtypeStruct((n, _MAX_PEAKS, _MAX_PEAKS), jnp.float32),
        ],
    )(pxa, pya, sa, va, pxb, pyb, sb, vb, t16)


def _run_sc(paf2, lin, ux, uy, vm, at):
    n = lin.shape[0]
    npair = _MAX_PEAKS * _MAX_PEAKS
    mesh = plsc.VectorSubcoreMesh(core_axis_name="c", subcore_axis_name="s")
    fn = functools.partial(
        pl.kernel,
        out_type=jax.ShapeDtypeStruct((n, npair), jnp.float32),
        mesh=mesh,
        compiler_params=pltpu.CompilerParams(needs_layout_passes=False),
        scratch_types=[
            pltpu.VMEM((_H * _W,), jnp.float32),
            pltpu.VMEM((_H * _W,), jnp.float32),
            pltpu.VMEM((_N_SAMPLES, npair), jnp.int32),
            pltpu.VMEM((npair,), jnp.float32),
            pltpu.VMEM((npair,), jnp.float32),
            pltpu.VMEM((npair,), jnp.float32),
            pltpu.VMEM((npair,), jnp.float32),
            pltpu.VMEM((npair,), jnp.float32),
        ],
    )(_sc_body)
    return fn(paf2, lin, ux, uy, vm, at)


def kernel(heat_pred, paf_pred):
    B, K, H, W = heat_pred.shape
    h3 = heat_pred.reshape(B * K, H, W)
    px3, py3, scr3, val3 = _run_peaks(h3)
    px = px3.reshape(B, K, _MAX_PEAKS)
    py = py3.reshape(B, K, _MAX_PEAKS)
    scr = scr3.reshape(B, K, _MAX_PEAKS)
    valf = val3.reshape(B, K, _MAX_PEAKS)

    a_idx = jnp.asarray(_SKEL[:, 0])
    b_idx = jnp.asarray(_SKEL[:, 1])
    C = _SKEL.shape[0]
    n = B * C
    pxa = px[:, a_idx].reshape(n, _MAX_PEAKS)
    pya = py[:, a_idx].reshape(n, _MAX_PEAKS)
    sa = scr[:, a_idx].reshape(n, _MAX_PEAKS)
    va = valf[:, a_idx].reshape(n, _MAX_PEAKS)
    pxb = px[:, b_idx].reshape(n, _MAX_PEAKS)
    pyb = py[:, b_idx].reshape(n, _MAX_PEAKS)
    sb = scr[:, b_idx].reshape(n, _MAX_PEAKS)
    vb = valf[:, b_idx].reshape(n, _MAX_PEAKS)

    t = jnp.linspace(0.0, 1.0, _N_SAMPLES).astype(jnp.float32)
    t16 = jnp.zeros((1, 1, 16), jnp.float32).at[0, 0, :_N_SAMPLES].set(t)

    lin, ux, uy, vm, at = _run_geom(pxa, pya, sa, va, pxb, pyb, sb, vb, t16)
    npair = _MAX_PEAKS * _MAX_PEAKS
    lin = lin.reshape(n, _N_SAMPLES, npair)
    ux = ux.reshape(n, npair)
    uy = uy.reshape(n, npair)
    vm = vm.reshape(n, npair)
    at = at.reshape(n, npair)

    paf2 = paf_pred.reshape(B * paf_pred.shape[1], H * W)
    conn = _run_sc(paf2, lin, ux, uy, vm, at)
    conn_scores = conn.reshape(B, C, _MAX_PEAKS, _MAX_PEAKS)
    return (px, py, scr, conn_scores)
